# speedup vs baseline: 7.4138x; 7.4138x over previous
import jax
import jax.numpy as jnp
from jax.experimental import pallas as pl
from jax.experimental.pallas import tpu as pltpu
from jax.experimental.pallas import tpu_sc as plsc

EMBED = 128
WINDOW = 128


def kernel(x, table):
    B, L = x.shape
    n = B * L
    idx = x.reshape(1, n).astype(jnp.int32)

    mesh = plsc.VectorSubcoreMesh(core_axis_name="c", subcore_axis_name="s")

    @pl.kernel(
        out_type=jax.ShapeDtypeStruct((n, EMBED), table.dtype),
        mesh=mesh,
    )
    def gather_kernel(table_hbm, idx_hbm, out_hbm):
        def body(i_vmem, o_vmem):
            pltpu.sync_copy(table_hbm.at[i_vmem.at[0]], o_vmem)

        pltpu.emit_pipeline(
            body,
            grid=(n // WINDOW,),
            in_specs=[pl.BlockSpec((1, WINDOW), index_map=lambda i: (0, i))],
            out_specs=[pl.BlockSpec((WINDOW, EMBED), index_map=lambda i: (i, 0))],
            core_axis_name=("c", "s"),
            dimension_semantics=(pltpu.PARALLEL,),
        )(idx_hbm, out_hbm)

    out = gather_kernel(table, idx)
    return out.reshape(B, L, EMBED)


# speedup vs baseline: 9.1569x; 1.2351x over previous
import jax
import jax.numpy as jnp
from jax.experimental import pallas as pl
from jax.experimental.pallas import tpu as pltpu
from jax.experimental.pallas import tpu_sc as plsc

EMBED = 128
WINDOW = 256


def kernel(x, table):
    B, L = x.shape
    n = B * L
    idx = x.reshape(1, n).astype(jnp.int32)

    mesh = plsc.VectorSubcoreMesh(core_axis_name="c", subcore_axis_name="s")

    @pl.kernel(
        out_type=jax.ShapeDtypeStruct((n, EMBED), table.dtype),
        mesh=mesh,
    )
    def gather_kernel(table_hbm, idx_hbm, out_hbm):
        def body(i_vmem, o_vmem):
            pltpu.sync_copy(table_hbm.at[i_vmem.at[0]], o_vmem)

        pltpu.emit_pipeline(
            body,
            grid=(n // WINDOW,),
            in_specs=[pl.BlockSpec((1, WINDOW), index_map=lambda i: (0, i))],
            out_specs=[pl.BlockSpec((WINDOW, EMBED), index_map=lambda i: (i, 0))],
            core_axis_name=("c", "s"),
            dimension_semantics=(pltpu.PARALLEL,),
        )(idx_hbm, out_hbm)

    out = gather_kernel(table, idx)
    return out.reshape(B, L, EMBED)


# speedup vs baseline: 9.2385x; 1.0089x over previous
import functools

import jax
import jax.numpy as jnp
from jax import lax
from jax.experimental import pallas as pl
from jax.experimental.pallas import tpu as pltpu
from jax.experimental.pallas import tpu_sc as plsc

EMBED = 128
NC = 2
NS = 16
NW = NC * NS
CH = 256


def kernel(x, table):
    B, L = x.shape
    n = B * L
    per_w = n // NW
    nch = per_w // CH
    idx = x.reshape(NW, nch, CH).astype(jnp.int32)

    mesh = plsc.VectorSubcoreMesh(core_axis_name="c", subcore_axis_name="s")

    @functools.partial(
        pl.kernel,
        out_type=jax.ShapeDtypeStruct((n, EMBED), table.dtype),
        mesh=mesh,
        scratch_types=[
            pltpu.VMEM((CH,), jnp.int32),
            pltpu.VMEM((CH,), jnp.int32),
            pltpu.VMEM((CH, EMBED), jnp.float32),
            pltpu.VMEM((CH, EMBED), jnp.float32),
            pltpu.SemaphoreType.DMA,
            pltpu.SemaphoreType.DMA,
            pltpu.SemaphoreType.DMA,
            pltpu.SemaphoreType.DMA,
            pltpu.SemaphoreType.DMA,
            pltpu.SemaphoreType.DMA,
        ],
    )
    def gather_kernel(table_hbm, idx_hbm, out_hbm, idxc_a, idxc_b,
                      buf_a, buf_b, is_a, is_b, gs_a, gs_b, ws_a, ws_b):
        wid = lax.axis_index("s") * NC + lax.axis_index("c")
        base = wid * per_w

        def idx_start(i, idxc, sem):
            pltpu.async_copy(idx_hbm.at[wid, i], idxc, sem)

        def idx_wait(i, idxc, sem):
            pltpu.make_async_copy(idx_hbm.at[wid, i], idxc, sem).wait()

        def gather_start(idxc, buf, sem):
            pltpu.async_copy(table_hbm.at[idxc], buf, sem)

        def gather_wait(idxc, buf, sem):
            pltpu.make_async_copy(table_hbm.at[idxc], buf, sem).wait()

        def write_start(i, buf, sem):
            pltpu.async_copy(buf, out_hbm.at[pl.ds(base + i * CH, CH)], sem)

        def write_wait(i, buf, sem):
            pltpu.make_async_copy(
                buf, out_hbm.at[pl.ds(base + i * CH, CH)], sem
            ).wait()

        pltpu.sync_copy(idx_hbm.at[wid, 0], idxc_a)
        gather_start(idxc_a, buf_a, gs_a)
        idx_start(1, idxc_b, is_b)

        @pl.loop(0, nch // 2)
        def _(it):
            i = 2 * it
            gather_wait(idxc_a, buf_a, gs_a)
            write_start(i, buf_a, ws_a)

            @pl.when(i + 2 < nch)
            def _():
                idx_start(i + 2, idxc_a, is_a)

            @pl.when(it > 0)
            def _():
                write_wait(i - 1, buf_b, ws_b)

            idx_wait(i + 1, idxc_b, is_b)
            gather_start(idxc_b, buf_b, gs_b)
            gather_wait(idxc_b, buf_b, gs_b)
            write_start(i + 1, buf_b, ws_b)

            @pl.when(i + 3 < nch)
            def _():
                idx_start(i + 3, idxc_b, is_b)

            write_wait(i, buf_a, ws_a)

            @pl.when(i + 2 < nch)
            def _():
                idx_wait(i + 2, idxc_a, is_a)
                gather_start(idxc_a, buf_a, gs_a)

        write_wait(nch - 1, buf_b, ws_b)

    out = gather_kernel(table, idx)
    return out.reshape(B, L, EMBED)
